# Initial kernel scaffold; baseline (speedup 1.0000x reference)
import jax
import jax.numpy as jnp
from jax.experimental import pallas as pl


def kernel(x, edge_index, W1, b1, W2, b2):
    raise NotImplementedError("write your pallas kernel here")


# speedup vs baseline: 21.5795x; 21.5795x over previous
import functools

import jax
import jax.numpy as jnp
from jax import lax
from jax.experimental import pallas as pl
from jax.experimental.pallas import tpu as pltpu
from jax.experimental.pallas import tpu_sc as plsc

N = 10000
E = 320000
D = 128
DH = D // 2
NC = 2
NS = 16
NW = NC * NS
K = 125
EPW = E // NW
NCHUNK = EPW // K
EPT = E // NS
NCHUNK2 = EPT // K
RPT = 624
TAIL0 = N - NS * RPT
CH = 208
NCH = RPT // CH

_MESH = plsc.VectorSubcoreMesh(core_axis_name="c", subcore_axis_name="s")


def _sc_degree(dst3, zeros_n, ones_k):
    @functools.partial(
        pl.kernel,
        out_type=jax.ShapeDtypeStruct((NC * N,), jnp.float32),
        mesh=_MESH,
        scratch_types=[
            pltpu.VMEM((NCHUNK, K), jnp.int32),
            pltpu.VMEM((K,), jnp.float32),
            pltpu.VMEM((RPT,), jnp.float32),
            pltpu.VMEM_SHARED((N,), jnp.float32),
        ],
    )
    def deg_kernel(dst_hbm, zeros_hbm, ones_hbm, out_hbm, dst_v, ones_v,
                   zbuf, acc_sh):
        c = lax.axis_index("c")
        s = lax.axis_index("s")
        wid = c * NS + s
        pltpu.sync_copy(zeros_hbm, zbuf)
        pltpu.sync_copy(zbuf, acc_sh.at[pl.ds(s * RPT, RPT)])

        @pl.when(s == 0)
        def _():
            pltpu.sync_copy(zbuf.at[pl.ds(0, TAIL0)],
                            acc_sh.at[pl.ds(NS * RPT, TAIL0)])

        pltpu.sync_copy(dst_hbm.at[wid], dst_v)
        pltpu.sync_copy(ones_hbm, ones_v)
        plsc.subcore_barrier()

        @pl.loop(0, NCHUNK)
        def _(j):
            pltpu.sync_copy(ones_v, acc_sh.at[dst_v.at[j]], add=True)

        plsc.subcore_barrier()
        pltpu.sync_copy(acc_sh.at[pl.ds(s * RPT, RPT)], zbuf)
        pltpu.sync_copy(zbuf, out_hbm.at[pl.ds(c * N + s * RPT, RPT)])

        @pl.when(s == 0)
        def _():
            pltpu.sync_copy(acc_sh.at[pl.ds(NS * RPT, TAIL0)],
                            ones_v.at[pl.ds(0, TAIL0)])
            pltpu.sync_copy(ones_v.at[pl.ds(0, TAIL0)],
                            out_hbm.at[pl.ds(c * N + NS * RPT, TAIL0)])

    return deg_kernel(dst3, zeros_n, ones_k)


def _sc_scatter(yL, yR, src3, dst3, zeros_ch):
    @functools.partial(
        pl.kernel,
        out_type=jax.ShapeDtypeStruct((NC, N, DH), jnp.float32),
        mesh=_MESH,
        scratch_types=[
            pltpu.VMEM((NCHUNK2, K), jnp.int32),
            pltpu.VMEM((NCHUNK2, K), jnp.int32),
            pltpu.VMEM((K, DH), jnp.float32),
            pltpu.VMEM((K, DH), jnp.float32),
            pltpu.VMEM((CH, DH), jnp.float32),
            pltpu.VMEM_SHARED((N, DH), jnp.float32),
            pltpu.SemaphoreType.DMA,
            pltpu.SemaphoreType.DMA,
        ],
        compiler_params=pltpu.CompilerParams(use_tc_tiling_on_sc=False),
    )
    def scat_kernel(yl_hbm, yr_hbm, src_hbm, dst_hbm, zeros_hbm, out_hbm,
                    src_v, dst_v, rows0, rows1, zb, acc_sh, gsem0, gsem1):
        c = lax.axis_index("c")
        s = lax.axis_index("s")
        pltpu.sync_copy(zeros_hbm, zb)

        @pl.loop(0, NCH)
        def _(i):
            pltpu.sync_copy(zb, acc_sh.at[pl.ds(s * RPT + i * CH, CH)])

        @pl.when(s == 0)
        def _():
            pltpu.sync_copy(zb.at[pl.ds(0, TAIL0)],
                            acc_sh.at[pl.ds(NS * RPT, TAIL0)])

        pltpu.sync_copy(src_hbm.at[s], src_v)
        pltpu.sync_copy(dst_hbm.at[s], dst_v)
        plsc.subcore_barrier()

        def run(y_hbm):
            pltpu.async_copy(y_hbm.at[src_v.at[0]], rows0, gsem0)

            @pl.loop(0, NCHUNK2, step=2)
            def _(j):
                pltpu.make_async_copy(y_hbm.at[src_v.at[j]],
                                      rows0, gsem0).wait()
                pltpu.async_copy(y_hbm.at[src_v.at[j + 1]], rows1, gsem1)
                pltpu.sync_copy(rows0, acc_sh.at[dst_v.at[j]], add=True)
                pltpu.make_async_copy(y_hbm.at[src_v.at[j + 1]],
                                      rows1, gsem1).wait()

                @pl.when(j + 2 < NCHUNK2)
                def _():
                    pltpu.async_copy(y_hbm.at[src_v.at[j + 2]], rows0, gsem0)

                pltpu.sync_copy(rows1, acc_sh.at[dst_v.at[j + 1]], add=True)

        @pl.when(c == 0)
        def _():
            run(yl_hbm)

        @pl.when(c == 1)
        def _():
            run(yr_hbm)

        plsc.subcore_barrier()

        @pl.loop(0, NCH)
        def _(i):
            pltpu.sync_copy(acc_sh.at[pl.ds(s * RPT + i * CH, CH)], zb)
            pltpu.sync_copy(zb, out_hbm.at[c, pl.ds(s * RPT + i * CH, CH)])

        @pl.when(s == 0)
        def _():
            pltpu.sync_copy(acc_sh.at[pl.ds(NS * RPT, TAIL0)],
                            zb.at[pl.ds(0, TAIL0)])
            pltpu.sync_copy(zb.at[pl.ds(0, TAIL0)],
                            out_hbm.at[c, pl.ds(NS * RPT, TAIL0)])

    return scat_kernel(yL, yR, src3, dst3, zeros_ch)


BR = 2000


def _tc_pre(x, W1, d0, d1):
    def body(x_ref, w_ref, d0_ref, d1_ref, yl_ref, yr_ref, dinv_ref):
        dinv = lax.rsqrt(d0_ref[...] + d1_ref[...] + 1.0)
        xw = jnp.dot(x_ref[...], w_ref[...],
                     preferred_element_type=jnp.float32) * dinv
        yl_ref[...] = xw[:, :DH]
        yr_ref[...] = xw[:, DH:]
        dinv_ref[...] = dinv

    return pl.pallas_call(
        body,
        grid=(N // BR,),
        in_specs=[
            pl.BlockSpec((BR, D), lambda i: (i, 0)),
            pl.BlockSpec((D, D), lambda i: (0, 0)),
            pl.BlockSpec((BR, 1), lambda i: (i, 0)),
            pl.BlockSpec((BR, 1), lambda i: (i, 0)),
        ],
        out_specs=[
            pl.BlockSpec((BR, DH), lambda i: (i, 0)),
            pl.BlockSpec((BR, DH), lambda i: (i, 0)),
            pl.BlockSpec((BR, 1), lambda i: (i, 0)),
        ],
        out_shape=[
            jax.ShapeDtypeStruct((N, DH), jnp.float32),
            jax.ShapeDtypeStruct((N, DH), jnp.float32),
            jax.ShapeDtypeStruct((N, 1), jnp.float32),
        ],
    )(x, W1, d0, d1)


def _tc_mid(p, y1L, y1R, dinv, b1, W2):
    def body(p_ref, yl_ref, yr_ref, dinv_ref, b1_ref, w_ref,
             y2l_ref, y2r_ref):
        pre = jnp.concatenate(
            [p_ref[0] + yl_ref[...], p_ref[1] + yr_ref[...]], axis=1)
        pre = pre * dinv_ref[...] + b1_ref[...]
        h = jnp.maximum(pre, 0.0)
        y2 = jnp.dot(h, w_ref[...],
                     preferred_element_type=jnp.float32) * dinv_ref[...]
        y2l_ref[...] = y2[:, :DH]
        y2r_ref[...] = y2[:, DH:]

    return pl.pallas_call(
        body,
        grid=(N // BR,),
        in_specs=[
            pl.BlockSpec((NC, BR, DH), lambda i: (0, i, 0)),
            pl.BlockSpec((BR, DH), lambda i: (i, 0)),
            pl.BlockSpec((BR, DH), lambda i: (i, 0)),
            pl.BlockSpec((BR, 1), lambda i: (i, 0)),
            pl.BlockSpec((1, D), lambda i: (0, 0)),
            pl.BlockSpec((D, D), lambda i: (0, 0)),
        ],
        out_specs=[
            pl.BlockSpec((BR, DH), lambda i: (i, 0)),
            pl.BlockSpec((BR, DH), lambda i: (i, 0)),
        ],
        out_shape=[
            jax.ShapeDtypeStruct((N, DH), jnp.float32),
            jax.ShapeDtypeStruct((N, DH), jnp.float32),
        ],
    )(p, y1L, y1R, dinv, b1, W2)


def _tc_post(q, y2L, y2R, dinv, b2):
    def body(q_ref, yl_ref, yr_ref, dinv_ref, b2_ref, o_ref):
        acc = jnp.concatenate(
            [q_ref[0] + yl_ref[...], q_ref[1] + yr_ref[...]], axis=1)
        o_ref[...] = acc * dinv_ref[...] + b2_ref[...]

    return pl.pallas_call(
        body,
        grid=(N // BR,),
        in_specs=[
            pl.BlockSpec((NC, BR, DH), lambda i: (0, i, 0)),
            pl.BlockSpec((BR, DH), lambda i: (i, 0)),
            pl.BlockSpec((BR, DH), lambda i: (i, 0)),
            pl.BlockSpec((BR, 1), lambda i: (i, 0)),
            pl.BlockSpec((1, D), lambda i: (0, 0)),
        ],
        out_specs=pl.BlockSpec((BR, D), lambda i: (i, 0)),
        out_shape=jax.ShapeDtypeStruct((N, D), jnp.float32),
    )(q, y2L, y2R, dinv, b2)


def kernel(x, edge_index, W1, b1, W2, b2):
    src = edge_index[0].astype(jnp.int32)
    dst = edge_index[1].astype(jnp.int32)
    src3 = src.reshape(NS, NCHUNK2, K)
    dst3 = dst.reshape(NS, NCHUNK2, K)
    dst3d = dst.reshape(NW, NCHUNK, K)
    zeros_n = jnp.zeros((RPT,), jnp.float32)
    zeros_ch = jnp.zeros((CH, DH), jnp.float32)
    ones_k = jnp.ones((K,), jnp.float32)

    degp = _sc_degree(dst3d, zeros_n, ones_k).reshape(NC, N)
    d0 = degp[0].reshape(N, 1)
    d1 = degp[1].reshape(N, 1)

    y1L, y1R, dinv = _tc_pre(x, W1, d0, d1)
    p = _sc_scatter(y1L, y1R, src3, dst3, zeros_ch)
    y2L, y2R = _tc_mid(p, y1L, y1R, dinv, b1.reshape(1, D), W2)
    q = _sc_scatter(y2L, y2R, src3, dst3, zeros_ch)
    return _tc_post(q, y2L, y2R, dinv, b2.reshape(1, D))


# speedup vs baseline: 28.7826x; 1.3338x over previous
import functools

import jax
import jax.numpy as jnp
from jax import lax
from jax.experimental import pallas as pl
from jax.experimental.pallas import tpu as pltpu
from jax.experimental.pallas import tpu_sc as plsc

N = 10000
E = 320000
D = 128
DH = D // 2
NC = 2
NS = 16
NW = NC * NS
K = 125
EPW = E // NW
NCHUNK = EPW // K
EPT = E // NS
NCHUNK2 = EPT // K
RPT = 624
TAIL0 = N - NS * RPT
CH = 208
NCH = RPT // CH
NBUF = 4

_MESH = plsc.VectorSubcoreMesh(core_axis_name="c", subcore_axis_name="s")


def _sc_degree(dst3, zeros_n, ones_k):
    @functools.partial(
        pl.kernel,
        out_type=jax.ShapeDtypeStruct((NC * N,), jnp.float32),
        mesh=_MESH,
        scratch_types=[
            pltpu.VMEM((NCHUNK, K), jnp.int32),
            pltpu.VMEM((K,), jnp.float32),
            pltpu.VMEM((RPT,), jnp.float32),
            pltpu.VMEM_SHARED((N,), jnp.float32),
        ],
    )
    def deg_kernel(dst_hbm, zeros_hbm, ones_hbm, out_hbm, dst_v, ones_v,
                   zbuf, acc_sh):
        c = lax.axis_index("c")
        s = lax.axis_index("s")
        wid = c * NS + s
        pltpu.sync_copy(zeros_hbm, zbuf)
        pltpu.sync_copy(zbuf, acc_sh.at[pl.ds(s * RPT, RPT)])

        @pl.when(s == 0)
        def _():
            pltpu.sync_copy(zbuf.at[pl.ds(0, TAIL0)],
                            acc_sh.at[pl.ds(NS * RPT, TAIL0)])

        pltpu.sync_copy(dst_hbm.at[wid], dst_v)
        pltpu.sync_copy(ones_hbm, ones_v)
        plsc.subcore_barrier()

        @pl.loop(0, NCHUNK)
        def _(j):
            pltpu.sync_copy(ones_v, acc_sh.at[dst_v.at[j]], add=True)

        plsc.subcore_barrier()
        pltpu.sync_copy(acc_sh.at[pl.ds(s * RPT, RPT)], zbuf)
        pltpu.sync_copy(zbuf, out_hbm.at[pl.ds(c * N + s * RPT, RPT)])

        @pl.when(s == 0)
        def _():
            pltpu.sync_copy(acc_sh.at[pl.ds(NS * RPT, TAIL0)],
                            ones_v.at[pl.ds(0, TAIL0)])
            pltpu.sync_copy(ones_v.at[pl.ds(0, TAIL0)],
                            out_hbm.at[pl.ds(c * N + NS * RPT, TAIL0)])

    return deg_kernel(dst3, zeros_n, ones_k)


def _sc_scatter(yL, yR, src3, dst3, zeros_ch):
    @functools.partial(
        pl.kernel,
        out_type=jax.ShapeDtypeStruct((NC, N, DH), jnp.float32),
        mesh=_MESH,
        scratch_types=[
            pltpu.VMEM((NCHUNK2, K), jnp.int32),
            pltpu.VMEM((NCHUNK2, K), jnp.int32),
            [pltpu.VMEM((K, DH), jnp.float32)] * NBUF,
            pltpu.VMEM((CH, DH), jnp.float32),
            pltpu.VMEM_SHARED((N, DH), jnp.float32),
            [pltpu.SemaphoreType.DMA] * NBUF,
            [pltpu.SemaphoreType.DMA] * NBUF,
        ],
        compiler_params=pltpu.CompilerParams(use_tc_tiling_on_sc=False),
    )
    def scat_kernel(yl_hbm, yr_hbm, src_hbm, dst_hbm, zeros_hbm, out_hbm,
                    src_v, dst_v, rows, zb, acc_sh, gsems, ssems):
        c = lax.axis_index("c")
        s = lax.axis_index("s")
        pltpu.sync_copy(zeros_hbm, zb)

        @pl.loop(0, NCH)
        def _(i):
            pltpu.sync_copy(zb, acc_sh.at[pl.ds(s * RPT + i * CH, CH)])

        @pl.when(s == 0)
        def _():
            pltpu.sync_copy(zb.at[pl.ds(0, TAIL0)],
                            acc_sh.at[pl.ds(NS * RPT, TAIL0)])

        pltpu.sync_copy(src_hbm.at[s], src_v)
        pltpu.sync_copy(dst_hbm.at[s], dst_v)
        plsc.subcore_barrier()

        def run(y_hbm):
            for b in range(NBUF):
                pltpu.async_copy(y_hbm.at[src_v.at[b]], rows[b], gsems[b])

            @pl.loop(0, NCHUNK2, step=NBUF)
            def _(j):
                sdescs = []
                for b in range(NBUF):
                    pltpu.make_async_copy(y_hbm.at[src_v.at[j + b]],
                                          rows[b], gsems[b]).wait()
                    sdescs.append(pltpu.async_copy(
                        rows[b], acc_sh.at[dst_v.at[j + b]], ssems[b],
                        add=True))
                for b in range(NBUF):
                    sdescs[b].wait()

                    @pl.when(j + b + NBUF < NCHUNK2)
                    def _():
                        pltpu.async_copy(y_hbm.at[src_v.at[j + b + NBUF]],
                                         rows[b], gsems[b])

        @pl.when(c == 0)
        def _():
            run(yl_hbm)

        @pl.when(c == 1)
        def _():
            run(yr_hbm)

        plsc.subcore_barrier()

        @pl.loop(0, NCH)
        def _(i):
            pltpu.sync_copy(acc_sh.at[pl.ds(s * RPT + i * CH, CH)], zb)
            pltpu.sync_copy(zb, out_hbm.at[c, pl.ds(s * RPT + i * CH, CH)])

        @pl.when(s == 0)
        def _():
            pltpu.sync_copy(acc_sh.at[pl.ds(NS * RPT, TAIL0)],
                            zb.at[pl.ds(0, TAIL0)])
            pltpu.sync_copy(zb.at[pl.ds(0, TAIL0)],
                            out_hbm.at[c, pl.ds(NS * RPT, TAIL0)])

    return scat_kernel(yL, yR, src3, dst3, zeros_ch)


BR = 2000


def _tc_pre(x, W1, d0, d1):
    def body(x_ref, w_ref, d0_ref, d1_ref, yl_ref, yr_ref, dinv_ref):
        dinv = lax.rsqrt(d0_ref[...] + d1_ref[...] + 1.0)
        xw = jnp.dot(x_ref[...], w_ref[...],
                     preferred_element_type=jnp.float32) * dinv
        yl_ref[...] = xw[:, :DH]
        yr_ref[...] = xw[:, DH:]
        dinv_ref[...] = dinv

    return pl.pallas_call(
        body,
        grid=(N // BR,),
        in_specs=[
            pl.BlockSpec((BR, D), lambda i: (i, 0)),
            pl.BlockSpec((D, D), lambda i: (0, 0)),
            pl.BlockSpec((BR, 1), lambda i: (i, 0)),
            pl.BlockSpec((BR, 1), lambda i: (i, 0)),
        ],
        out_specs=[
            pl.BlockSpec((BR, DH), lambda i: (i, 0)),
            pl.BlockSpec((BR, DH), lambda i: (i, 0)),
            pl.BlockSpec((BR, 1), lambda i: (i, 0)),
        ],
        out_shape=[
            jax.ShapeDtypeStruct((N, DH), jnp.float32),
            jax.ShapeDtypeStruct((N, DH), jnp.float32),
            jax.ShapeDtypeStruct((N, 1), jnp.float32),
        ],
    )(x, W1, d0, d1)


def _tc_mid(p, y1L, y1R, dinv, b1, W2):
    def body(p_ref, yl_ref, yr_ref, dinv_ref, b1_ref, w_ref,
             y2l_ref, y2r_ref):
        pre = jnp.concatenate(
            [p_ref[0] + yl_ref[...], p_ref[1] + yr_ref[...]], axis=1)
        pre = pre * dinv_ref[...] + b1_ref[...]
        h = jnp.maximum(pre, 0.0)
        y2 = jnp.dot(h, w_ref[...],
                     preferred_element_type=jnp.float32) * dinv_ref[...]
        y2l_ref[...] = y2[:, :DH]
        y2r_ref[...] = y2[:, DH:]

    return pl.pallas_call(
        body,
        grid=(N // BR,),
        in_specs=[
            pl.BlockSpec((NC, BR, DH), lambda i: (0, i, 0)),
            pl.BlockSpec((BR, DH), lambda i: (i, 0)),
            pl.BlockSpec((BR, DH), lambda i: (i, 0)),
            pl.BlockSpec((BR, 1), lambda i: (i, 0)),
            pl.BlockSpec((1, D), lambda i: (0, 0)),
            pl.BlockSpec((D, D), lambda i: (0, 0)),
        ],
        out_specs=[
            pl.BlockSpec((BR, DH), lambda i: (i, 0)),
            pl.BlockSpec((BR, DH), lambda i: (i, 0)),
        ],
        out_shape=[
            jax.ShapeDtypeStruct((N, DH), jnp.float32),
            jax.ShapeDtypeStruct((N, DH), jnp.float32),
        ],
    )(p, y1L, y1R, dinv, b1, W2)


def _tc_post(q, y2L, y2R, dinv, b2):
    def body(q_ref, yl_ref, yr_ref, dinv_ref, b2_ref, o_ref):
        acc = jnp.concatenate(
            [q_ref[0] + yl_ref[...], q_ref[1] + yr_ref[...]], axis=1)
        o_ref[...] = acc * dinv_ref[...] + b2_ref[...]

    return pl.pallas_call(
        body,
        grid=(N // BR,),
        in_specs=[
            pl.BlockSpec((NC, BR, DH), lambda i: (0, i, 0)),
            pl.BlockSpec((BR, DH), lambda i: (i, 0)),
            pl.BlockSpec((BR, DH), lambda i: (i, 0)),
            pl.BlockSpec((BR, 1), lambda i: (i, 0)),
            pl.BlockSpec((1, D), lambda i: (0, 0)),
        ],
        out_specs=pl.BlockSpec((BR, D), lambda i: (i, 0)),
        out_shape=jax.ShapeDtypeStruct((N, D), jnp.float32),
    )(q, y2L, y2R, dinv, b2)


def kernel(x, edge_index, W1, b1, W2, b2):
    src = edge_index[0].astype(jnp.int32)
    dst = edge_index[1].astype(jnp.int32)
    src3 = src.reshape(NS, NCHUNK2, K)
    dst3 = dst.reshape(NS, NCHUNK2, K)
    dst3d = dst.reshape(NW, NCHUNK, K)
    zeros_n = jnp.zeros((RPT,), jnp.float32)
    zeros_ch = jnp.zeros((CH, DH), jnp.float32)
    ones_k = jnp.ones((K,), jnp.float32)

    degp = _sc_degree(dst3d, zeros_n, ones_k).reshape(NC, N)
    d0 = degp[0].reshape(N, 1)
    d1 = degp[1].reshape(N, 1)

    y1L, y1R, dinv = _tc_pre(x, W1, d0, d1)
    p = _sc_scatter(y1L, y1R, src3, dst3, zeros_ch)
    y2L, y2R = _tc_mid(p, y1L, y1R, dinv, b1.reshape(1, D), W2)
    q = _sc_scatter(y2L, y2R, src3, dst3, zeros_ch)
    return _tc_post(q, y2L, y2R, dinv, b2.reshape(1, D))


# speedup vs baseline: 29.4039x; 1.0216x over previous
import functools

import jax
import jax.numpy as jnp
from jax import lax
from jax.experimental import pallas as pl
from jax.experimental.pallas import tpu as pltpu
from jax.experimental.pallas import tpu_sc as plsc

N = 10000
E = 320000
D = 128
DH = D // 2
NC = 2
NS = 16
NW = NC * NS
K = 125
EPW = E // NW
NCHUNK = EPW // K
EPT = E // NS
NCHUNK2 = EPT // K
RPT = 624
TAIL0 = N - NS * RPT
CH = 208
NCH = RPT // CH
NBUF = 8
HNCH = NCHUNK2 // 2
WCH = 104
NWCH = RPT // WCH

_MESH = plsc.VectorSubcoreMesh(core_axis_name="c", subcore_axis_name="s")


def _sc_degree(dst3, zeros_n, ones_k):
    @functools.partial(
        pl.kernel,
        out_type=jax.ShapeDtypeStruct((NC * N,), jnp.float32),
        mesh=_MESH,
        scratch_types=[
            pltpu.VMEM((NCHUNK, K), jnp.int32),
            pltpu.VMEM((K,), jnp.float32),
            pltpu.VMEM((RPT,), jnp.float32),
            pltpu.VMEM_SHARED((N,), jnp.float32),
        ],
    )
    def deg_kernel(dst_hbm, zeros_hbm, ones_hbm, out_hbm, dst_v, ones_v,
                   zbuf, acc_sh):
        c = lax.axis_index("c")
        s = lax.axis_index("s")
        wid = c * NS + s
        pltpu.sync_copy(zeros_hbm, zbuf)
        pltpu.sync_copy(zbuf, acc_sh.at[pl.ds(s * RPT, RPT)])

        @pl.when(s == 0)
        def _():
            pltpu.sync_copy(zbuf.at[pl.ds(0, TAIL0)],
                            acc_sh.at[pl.ds(NS * RPT, TAIL0)])

        pltpu.sync_copy(dst_hbm.at[wid], dst_v)
        pltpu.sync_copy(ones_hbm, ones_v)
        plsc.subcore_barrier()

        @pl.loop(0, NCHUNK)
        def _(j):
            pltpu.sync_copy(ones_v, acc_sh.at[dst_v.at[j]], add=True)

        plsc.subcore_barrier()
        pltpu.sync_copy(acc_sh.at[pl.ds(s * RPT, RPT)], zbuf)
        pltpu.sync_copy(zbuf, out_hbm.at[pl.ds(c * N + s * RPT, RPT)])

        @pl.when(s == 0)
        def _():
            pltpu.sync_copy(acc_sh.at[pl.ds(NS * RPT, TAIL0)],
                            ones_v.at[pl.ds(0, TAIL0)])
            pltpu.sync_copy(ones_v.at[pl.ds(0, TAIL0)],
                            out_hbm.at[pl.ds(c * N + NS * RPT, TAIL0)])

    return deg_kernel(dst3, zeros_n, ones_k)


def _sc_scatter(yL, yR, src3, dst3, zeros_ch):
    @functools.partial(
        pl.kernel,
        out_type=jax.ShapeDtypeStruct((NC, N, DH), jnp.float32),
        mesh=_MESH,
        scratch_types=[
            pltpu.VMEM((HNCH, K), jnp.int32),
            pltpu.VMEM((HNCH, K), jnp.int32),
            [pltpu.VMEM((K, DH), jnp.float32)] * NBUF,
            pltpu.VMEM_SHARED((N, DH), jnp.float32),
            [pltpu.SemaphoreType.DMA] * NBUF,
            [pltpu.SemaphoreType.DMA] * NBUF,
        ],
        compiler_params=pltpu.CompilerParams(use_tc_tiling_on_sc=False),
    )
    def scat_kernel(yl_hbm, yr_hbm, src_hbm, dst_hbm, zeros_hbm, out_hbm,
                    src_v, dst_v, rows, acc_sh, gsems, ssems):
        c = lax.axis_index("c")
        s = lax.axis_index("s")
        pltpu.sync_copy(zeros_hbm, rows[0])

        @pl.loop(0, NWCH)
        def _(i):
            pltpu.sync_copy(rows[0].at[pl.ds(0, WCH)],
                            acc_sh.at[pl.ds(s * RPT + i * WCH, WCH)])

        @pl.when(s == 0)
        def _():
            pltpu.sync_copy(rows[0].at[pl.ds(0, TAIL0)],
                            acc_sh.at[pl.ds(NS * RPT, TAIL0)])

        plsc.subcore_barrier()

        def run(y_hbm):
            for h in range(2):
                pltpu.sync_copy(src_hbm.at[s, pl.ds(h * HNCH, HNCH)], src_v)
                pltpu.sync_copy(dst_hbm.at[s, pl.ds(h * HNCH, HNCH)], dst_v)
                for b in range(NBUF):
                    pltpu.async_copy(y_hbm.at[src_v.at[b]], rows[b],
                                     gsems[b])

                @pl.loop(0, HNCH, step=NBUF)
                def _(j):
                    sdescs = []
                    for b in range(NBUF):
                        pltpu.make_async_copy(y_hbm.at[src_v.at[j + b]],
                                              rows[b], gsems[b]).wait()
                        sdescs.append(pltpu.async_copy(
                            rows[b], acc_sh.at[dst_v.at[j + b]], ssems[b],
                            add=True))
                    for b in range(NBUF):
                        sdescs[b].wait()

                        @pl.when(j + b + NBUF < HNCH)
                        def _():
                            pltpu.async_copy(
                                y_hbm.at[src_v.at[j + b + NBUF]],
                                rows[b], gsems[b])

        @pl.when(c == 0)
        def _():
            run(yl_hbm)

        @pl.when(c == 1)
        def _():
            run(yr_hbm)

        plsc.subcore_barrier()

        @pl.loop(0, NWCH)
        def _(i):
            pltpu.sync_copy(acc_sh.at[pl.ds(s * RPT + i * WCH, WCH)],
                            rows[0].at[pl.ds(0, WCH)])
            pltpu.sync_copy(rows[0].at[pl.ds(0, WCH)],
                            out_hbm.at[c, pl.ds(s * RPT + i * WCH, WCH)])

        @pl.when(s == 0)
        def _():
            pltpu.sync_copy(acc_sh.at[pl.ds(NS * RPT, TAIL0)],
                            rows[1].at[pl.ds(0, TAIL0)])
            pltpu.sync_copy(rows[1].at[pl.ds(0, TAIL0)],
                            out_hbm.at[c, pl.ds(NS * RPT, TAIL0)])

    return scat_kernel(yL, yR, src3, dst3, zeros_ch)


BR = 2000


def _tc_pre(x, W1, d0, d1):
    def body(x_ref, w_ref, d0_ref, d1_ref, yl_ref, yr_ref, dinv_ref):
        dinv = lax.rsqrt(d0_ref[...] + d1_ref[...] + 1.0)
        xw = jnp.dot(x_ref[...], w_ref[...],
                     preferred_element_type=jnp.float32) * dinv
        yl_ref[...] = xw[:, :DH]
        yr_ref[...] = xw[:, DH:]
        dinv_ref[...] = dinv

    return pl.pallas_call(
        body,
        grid=(N // BR,),
        in_specs=[
            pl.BlockSpec((BR, D), lambda i: (i, 0)),
            pl.BlockSpec((D, D), lambda i: (0, 0)),
            pl.BlockSpec((BR, 1), lambda i: (i, 0)),
            pl.BlockSpec((BR, 1), lambda i: (i, 0)),
        ],
        out_specs=[
            pl.BlockSpec((BR, DH), lambda i: (i, 0)),
            pl.BlockSpec((BR, DH), lambda i: (i, 0)),
            pl.BlockSpec((BR, 1), lambda i: (i, 0)),
        ],
        out_shape=[
            jax.ShapeDtypeStruct((N, DH), jnp.float32),
            jax.ShapeDtypeStruct((N, DH), jnp.float32),
            jax.ShapeDtypeStruct((N, 1), jnp.float32),
        ],
    )(x, W1, d0, d1)


def _tc_mid(p, y1L, y1R, dinv, b1, W2):
    def body(p_ref, yl_ref, yr_ref, dinv_ref, b1_ref, w_ref,
             y2l_ref, y2r_ref):
        pre = jnp.concatenate(
            [p_ref[0] + yl_ref[...], p_ref[1] + yr_ref[...]], axis=1)
        pre = pre * dinv_ref[...] + b1_ref[...]
        h = jnp.maximum(pre, 0.0)
        y2 = jnp.dot(h, w_ref[...],
                     preferred_element_type=jnp.float32) * dinv_ref[...]
        y2l_ref[...] = y2[:, :DH]
        y2r_ref[...] = y2[:, DH:]

    return pl.pallas_call(
        body,
        grid=(N // BR,),
        in_specs=[
            pl.BlockSpec((NC, BR, DH), lambda i: (0, i, 0)),
            pl.BlockSpec((BR, DH), lambda i: (i, 0)),
            pl.BlockSpec((BR, DH), lambda i: (i, 0)),
            pl.BlockSpec((BR, 1), lambda i: (i, 0)),
            pl.BlockSpec((1, D), lambda i: (0, 0)),
            pl.BlockSpec((D, D), lambda i: (0, 0)),
        ],
        out_specs=[
            pl.BlockSpec((BR, DH), lambda i: (i, 0)),
            pl.BlockSpec((BR, DH), lambda i: (i, 0)),
        ],
        out_shape=[
            jax.ShapeDtypeStruct((N, DH), jnp.float32),
            jax.ShapeDtypeStruct((N, DH), jnp.float32),
        ],
    )(p, y1L, y1R, dinv, b1, W2)


def _tc_post(q, y2L, y2R, dinv, b2):
    def body(q_ref, yl_ref, yr_ref, dinv_ref, b2_ref, o_ref):
        acc = jnp.concatenate(
            [q_ref[0] + yl_ref[...], q_ref[1] + yr_ref[...]], axis=1)
        o_ref[...] = acc * dinv_ref[...] + b2_ref[...]

    return pl.pallas_call(
        body,
        grid=(N // BR,),
        in_specs=[
            pl.BlockSpec((NC, BR, DH), lambda i: (0, i, 0)),
            pl.BlockSpec((BR, DH), lambda i: (i, 0)),
            pl.BlockSpec((BR, DH), lambda i: (i, 0)),
            pl.BlockSpec((BR, 1), lambda i: (i, 0)),
            pl.BlockSpec((1, D), lambda i: (0, 0)),
        ],
        out_specs=pl.BlockSpec((BR, D), lambda i: (i, 0)),
        out_shape=jax.ShapeDtypeStruct((N, D), jnp.float32),
    )(q, y2L, y2R, dinv, b2)


def kernel(x, edge_index, W1, b1, W2, b2):
    src = edge_index[0].astype(jnp.int32)
    dst = edge_index[1].astype(jnp.int32)
    src3 = src.reshape(NS, NCHUNK2, K)
    dst3 = dst.reshape(NS, NCHUNK2, K)
    dst3d = dst.reshape(NW, NCHUNK, K)
    zeros_n = jnp.zeros((RPT,), jnp.float32)
    zeros_ch = jnp.zeros((K, DH), jnp.float32)
    ones_k = jnp.ones((K,), jnp.float32)

    degp = _sc_degree(dst3d, zeros_n, ones_k).reshape(NC, N)
    d0 = degp[0].reshape(N, 1)
    d1 = degp[1].reshape(N, 1)

    y1L, y1R, dinv = _tc_pre(x, W1, d0, d1)
    p = _sc_scatter(y1L, y1R, src3, dst3, zeros_ch)
    y2L, y2R = _tc_mid(p, y1L, y1R, dinv, b1.reshape(1, D), W2)
    q = _sc_scatter(y2L, y2R, src3, dst3, zeros_ch)
    return _tc_post(q, y2L, y2R, dinv, b2.reshape(1, D))
